# baseline (speedup 1.0000x reference)
import functools

import jax
import jax.numpy as jnp
from jax import lax
from jax.experimental import pallas as pl
from jax.experimental.pallas import tpu as pltpu
from jax.experimental.pallas import tpu_sc as plsc

BATCH = 4096
D = 1000
DP = 1024
L = 16
NC, NS = 2, 16
NW = NC * NS
BPW = BATCH // NW
C = 8
NCHUNK = BPW // C
NV = D // L
TAIL = D - NV * L
DM = 896
NGRP = 100000 // 8

_GATHER_DNUMS = lax.GatherDimensionNumbers(
    offset_dims=(), collapsed_slice_dims=(0,), start_index_map=(0,))


def _permute(v, idx):
    return lax.gather(v, idx[:, None], _GATHER_DNUMS, (1,),
                      mode=lax.GatherScatterMode.PROMISE_IN_BOUNDS)


def _sc_body(a_hbm, at3_hbm, w_hbm, u_hbm, i_hbm, j_hbm, oi_hbm, oj_hbm,
             u_v, g_v, i_v, j_v, a_v, at_v, wi_v, wj_v, oi_v, oj_v,
             sem_a, sem_w):
    wid = lax.axis_index("s") * NC + lax.axis_index("c")
    base = wid * BPW
    pltpu.sync_copy(u_hbm.at[pl.ds(base, BPW)], u_v.at[pl.ds(0, BPW)])
    pltpu.sync_copy(i_hbm.at[pl.ds(base, BPW)], i_v)
    pltpu.sync_copy(j_hbm.at[pl.ds(base, BPW)], j_v)

    lane = lax.iota(jnp.int32, L)
    tail_mask = lane >= (L - TAIL)
    zero = jnp.zeros((L,), jnp.float32)

    def g_fn(q, _):
        uw = u_v[pl.ds(q * L, L)]
        g_v[pl.ds(q * L, L)] = lax.shift_right_logical(uw, 3)
        return 0

    lax.fori_loop(0, BPW // L, g_fn, 0)

    def chunk_fn(c, chunk_res):
        off = c * C
        parity = lax.rem(c, 2)
        cpw1 = pltpu.async_copy(w_hbm.at[i_v.at[pl.ds(off, C)]], wi_v, sem_w)
        cpw2 = pltpu.async_copy(w_hbm.at[j_v.at[pl.ds(off, C)]], wj_v, sem_w)
        cpt = pltpu.async_copy(at3_hbm.at[g_v.at[pl.ds(off, C)]], at_v, sem_w)
        uvec = u_v[pl.ds(off, L)]
        us = [uvec[r] for r in range(C)]
        rms = [lax.rem(u, 8) for u in us]
        row_cps = []
        for r in range(C):
            g8 = pl.multiple_of(us[r] - rms[r], 8)
            cp = pltpu.async_copy(
                a_hbm.at[pl.ds(g8, 8), pl.ds(0, DM)], a_v.at[r], sem_a)
            row_cps.append(cp)
        for cp in row_cps:
            cp.wait()
        cpw1.wait()
        cpw2.wait()
        cpt.wait()

        res_i, res_j = chunk_res
        for r in range(C):
            rm = rms[r]

            def k_fn(s, acc, r=r, rm=rm):
                ai, aj = acc
                for t in range(8):
                    av = a_v[r, rm, pl.ds(s * 128 + t * L, L)]
                    ai = ai + av * wi_v[r, s, pl.ds(t * L, L)]
                    aj = aj + av * wj_v[r, s, pl.ds(t * L, L)]
                return ai, aj

            ai, aj = lax.fori_loop(0, 7, k_fn, (zero, zero))
            for t in range(6):
                av = at_v[r, rm, pl.ds(t * L, L)]
                ai = ai + av * wi_v[r, 7, pl.ds(t * L, L)]
                aj = aj + av * wj_v[r, 7, pl.ds(t * L, L)]
            av = at_v[r, rm, pl.ds(88, L)]
            ai = ai + jnp.where(tail_mask, av * wi_v[r, 7, pl.ds(88, L)], 0.0)
            aj = aj + jnp.where(tail_mask, av * wj_v[r, 7, pl.ds(88, L)], 0.0)
            for s in (8, 4, 2, 1):
                perm = lane ^ s
                ai = ai + _permute(ai, perm)
                aj = aj + _permute(aj, perm)
            sel = lane == (r + parity * C)
            res_i = jnp.where(sel, ai, res_i)
            res_j = jnp.where(sel, aj, res_j)

        @pl.when(parity == 1)
        def _store():
            st = (c - 1) * C
            oi_v[pl.ds(st, 2 * C)] = res_i
            oj_v[pl.ds(st, 2 * C)] = res_j

        keep = parity == 0
        return (jnp.where(keep, res_i, zero), jnp.where(keep, res_j, zero))

    lax.fori_loop(0, NCHUNK, chunk_fn, (zero, zero))
    pltpu.sync_copy(oi_v, oi_hbm.at[pl.ds(base, BPW)])
    pltpu.sync_copy(oj_v, oj_hbm.at[pl.ds(base, BPW)])


def kernel(A, W, user, item_i, item_j):
    user = user.astype(jnp.int32)
    item_i = item_i.astype(jnp.int32)
    item_j = item_j.astype(jnp.int32)
    w_pad = jnp.pad(W, ((0, 0), (0, DP - D))).reshape(D, 8, 128)
    a_tail = jnp.pad(A[:, DM:], ((0, 0), (0, 128 - (D - DM))))
    a_tail = a_tail.reshape(NGRP, 8, 128)
    mesh = plsc.VectorSubcoreMesh(core_axis_name="c", subcore_axis_name="s")
    f32 = jnp.float32
    run = pl.kernel(
        _sc_body,
        out_type=(jax.ShapeDtypeStruct((BATCH,), f32),
                  jax.ShapeDtypeStruct((BATCH,), f32)),
        mesh=mesh,
        scratch_types=[
            pltpu.VMEM((BPW + L - C,), jnp.int32),
            pltpu.VMEM((BPW,), jnp.int32),
            pltpu.VMEM((BPW,), jnp.int32),
            pltpu.VMEM((BPW,), jnp.int32),
            pltpu.VMEM((C, 8, DM), f32),
            pltpu.VMEM((C, 8, 128), f32),
            pltpu.VMEM((C, 8, 128), f32),
            pltpu.VMEM((C, 8, 128), f32),
            pltpu.VMEM((BPW,), f32),
            pltpu.VMEM((BPW,), f32),
            pltpu.SemaphoreType.DMA,
            pltpu.SemaphoreType.DMA,
        ],
    )
    return run(A, a_tail, w_pad, user, item_i, item_j)


# speedup vs baseline: 1.1012x; 1.1012x over previous
import functools

import jax
import jax.numpy as jnp
from jax import lax
from jax.experimental import pallas as pl
from jax.experimental.pallas import tpu as pltpu
from jax.experimental.pallas import tpu_sc as plsc

BATCH = 4096
D = 1000
DP = 1024
L = 16
NC, NS = 2, 16
NW = NC * NS
BPW = BATCH // NW
C = 8
NCHUNK = BPW // C
NV = D // L
TAIL = D - NV * L
DM = 896
NGRP = 100000 // 8

_GATHER_DNUMS = lax.GatherDimensionNumbers(
    offset_dims=(), collapsed_slice_dims=(0,), start_index_map=(0,))

TAIL_ROWS = 5000


def _tail_body(a_ref, o_ref):
    o_ref[...] = a_ref[...]


def _make_a_tail(A):
    out = pl.pallas_call(
        _tail_body,
        grid=(100000 // TAIL_ROWS,),
        in_specs=[pl.BlockSpec((TAIL_ROWS, 128), lambda i: (i, 7))],
        out_specs=pl.BlockSpec((TAIL_ROWS, 128), lambda i: (i, 0)),
        out_shape=jax.ShapeDtypeStruct((100000, 128), jnp.float32),
    )(A)
    return out.reshape(NGRP, 8, 128)


def _w3_body(w_ref, o_ref):
    for s in range(7):
        o_ref[:, s, :] = w_ref[:, pl.ds(s * 128, 128)]
    o_ref[:, 7, pl.ds(0, D - DM)] = w_ref[:, pl.ds(DM, D - DM)]


def _make_w3(W):
    return pl.pallas_call(
        _w3_body,
        out_shape=jax.ShapeDtypeStruct((D, 8, 128), jnp.float32),
    )(W)


def _permute(v, idx):
    return lax.gather(v, idx[:, None], _GATHER_DNUMS, (1,),
                      mode=lax.GatherScatterMode.PROMISE_IN_BOUNDS)


def _sc_body(a_hbm, at3_hbm, w_hbm, u_hbm, i_hbm, j_hbm, oi_hbm, oj_hbm,
             u_v, g_v, i_v, j_v, a_v, at_v, wi_v, wj_v, oi_v, oj_v,
             sem_a, sem_w):
    wid = lax.axis_index("s") * NC + lax.axis_index("c")
    base = wid * BPW
    pltpu.sync_copy(u_hbm.at[pl.ds(base, BPW)], u_v.at[pl.ds(0, BPW)])
    pltpu.sync_copy(i_hbm.at[pl.ds(base, BPW)], i_v)
    pltpu.sync_copy(j_hbm.at[pl.ds(base, BPW)], j_v)

    lane = lax.iota(jnp.int32, L)
    tail_mask = lane >= (L - TAIL)
    zero = jnp.zeros((L,), jnp.float32)

    def g_fn(q, _):
        uw = u_v[pl.ds(q * L, L)]
        g_v[pl.ds(q * L, L)] = lax.shift_right_logical(uw, 3)
        return 0

    lax.fori_loop(0, BPW // L, g_fn, 0)

    def chunk_fn(c, chunk_res):
        off = c * C
        parity = lax.rem(c, 2)
        cpw1 = pltpu.async_copy(w_hbm.at[i_v.at[pl.ds(off, C)]], wi_v, sem_w)
        cpw2 = pltpu.async_copy(w_hbm.at[j_v.at[pl.ds(off, C)]], wj_v, sem_w)
        cpt = pltpu.async_copy(at3_hbm.at[g_v.at[pl.ds(off, C)]], at_v, sem_w)
        uvec = u_v[pl.ds(off, L)]
        us = [uvec[r] for r in range(C)]
        rms = [lax.rem(u, 8) for u in us]
        row_cps = []
        for r in range(C):
            g8 = pl.multiple_of(us[r] - rms[r], 8)
            cp = pltpu.async_copy(
                a_hbm.at[pl.ds(g8, 8), pl.ds(0, DM)], a_v.at[r], sem_a)
            row_cps.append(cp)
        for cp in row_cps:
            cp.wait()
        cpw1.wait()
        cpw2.wait()
        cpt.wait()

        res_i, res_j = chunk_res
        for r in range(C):
            rm = rms[r]

            def k_fn(s, acc, r=r, rm=rm):
                ai, aj = acc
                for t in range(8):
                    av = a_v[r, rm, pl.ds(s * 128 + t * L, L)]
                    ai = ai + av * wi_v[r, s, pl.ds(t * L, L)]
                    aj = aj + av * wj_v[r, s, pl.ds(t * L, L)]
                return ai, aj

            ai, aj = lax.fori_loop(0, 7, k_fn, (zero, zero))
            for t in range(6):
                av = at_v[r, rm, pl.ds(t * L, L)]
                ai = ai + av * wi_v[r, 7, pl.ds(t * L, L)]
                aj = aj + av * wj_v[r, 7, pl.ds(t * L, L)]
            av = at_v[r, rm, pl.ds(88, L)]
            ai = ai + jnp.where(tail_mask, av * wi_v[r, 7, pl.ds(88, L)], 0.0)
            aj = aj + jnp.where(tail_mask, av * wj_v[r, 7, pl.ds(88, L)], 0.0)
            for s in (8, 4, 2, 1):
                perm = lane ^ s
                ai = ai + _permute(ai, perm)
                aj = aj + _permute(aj, perm)
            sel = lane == (r + parity * C)
            res_i = jnp.where(sel, ai, res_i)
            res_j = jnp.where(sel, aj, res_j)

        @pl.when(parity == 1)
        def _store():
            st = (c - 1) * C
            oi_v[pl.ds(st, 2 * C)] = res_i
            oj_v[pl.ds(st, 2 * C)] = res_j

        keep = parity == 0
        return (jnp.where(keep, res_i, zero), jnp.where(keep, res_j, zero))

    lax.fori_loop(0, NCHUNK, chunk_fn, (zero, zero))
    pltpu.sync_copy(oi_v, oi_hbm.at[pl.ds(base, BPW)])
    pltpu.sync_copy(oj_v, oj_hbm.at[pl.ds(base, BPW)])


def kernel(A, W, user, item_i, item_j):
    user = user.astype(jnp.int32)
    item_i = item_i.astype(jnp.int32)
    item_j = item_j.astype(jnp.int32)
    w_pad = _make_w3(W)
    a_tail = _make_a_tail(A)
    mesh = plsc.VectorSubcoreMesh(core_axis_name="c", subcore_axis_name="s")
    f32 = jnp.float32
    run = pl.kernel(
        _sc_body,
        out_type=(jax.ShapeDtypeStruct((BATCH,), f32),
                  jax.ShapeDtypeStruct((BATCH,), f32)),
        mesh=mesh,
        scratch_types=[
            pltpu.VMEM((BPW + L - C,), jnp.int32),
            pltpu.VMEM((BPW,), jnp.int32),
            pltpu.VMEM((BPW,), jnp.int32),
            pltpu.VMEM((BPW,), jnp.int32),
            pltpu.VMEM((C, 8, DM), f32),
            pltpu.VMEM((C, 8, 128), f32),
            pltpu.VMEM((C, 8, 128), f32),
            pltpu.VMEM((C, 8, 128), f32),
            pltpu.VMEM((BPW,), f32),
            pltpu.VMEM((BPW,), f32),
            pltpu.SemaphoreType.DMA,
            pltpu.SemaphoreType.DMA,
        ],
    )
    return run(A, a_tail, w_pad, user, item_i, item_j)


# speedup vs baseline: 1.1921x; 1.0825x over previous
import functools

import jax
import jax.numpy as jnp
from jax import lax
from jax.experimental import pallas as pl
from jax.experimental.pallas import tpu as pltpu
from jax.experimental.pallas import tpu_sc as plsc

BATCH = 4096
D = 1000
DP = 1024
L = 16
NC, NS = 2, 16
NW = NC * NS
BPW = BATCH // NW
C = 8
NCHUNK = BPW // C
NV = D // L
TAIL = D - NV * L
DM = 896
NGRP = 100000 // 8

_GATHER_DNUMS = lax.GatherDimensionNumbers(
    offset_dims=(), collapsed_slice_dims=(0,), start_index_map=(0,))

def _w3_body(w_ref, o_ref):
    for s in range(7):
        o_ref[:, s, :] = w_ref[:, pl.ds(s * 128, 128)]
    o_ref[:, 7, pl.ds(0, D - DM)] = w_ref[:, pl.ds(DM, D - DM)]


def _make_w3(W):
    return pl.pallas_call(
        _w3_body,
        out_shape=jax.ShapeDtypeStruct((D, 8, 128), jnp.float32),
    )(W)


def _permute(v, idx):
    return lax.gather(v, idx[:, None], _GATHER_DNUMS, (1,),
                      mode=lax.GatherScatterMode.PROMISE_IN_BOUNDS)


def _sc_body(a_hbm, w_hbm, u_hbm, i_hbm, j_hbm, oi_hbm, oj_hbm,
             u_v, i_v, j_v, a_v, at_v, wi_v, wj_v, oi_v, oj_v,
             sem_a, sem_w):
    wid = lax.axis_index("s") * NC + lax.axis_index("c")
    base = wid * BPW
    pltpu.sync_copy(u_hbm.at[pl.ds(base, BPW)], u_v.at[pl.ds(0, BPW)])
    pltpu.sync_copy(i_hbm.at[pl.ds(base, BPW)], i_v)
    pltpu.sync_copy(j_hbm.at[pl.ds(base, BPW)], j_v)

    lane = lax.iota(jnp.int32, L)
    tail_mask = lane >= (L - TAIL)
    zero = jnp.zeros((L,), jnp.float32)

    def chunk_fn(c, chunk_res):
        off = c * C
        parity = lax.rem(c, 2)
        cpw1 = pltpu.async_copy(w_hbm.at[i_v.at[pl.ds(off, C)]], wi_v, sem_w)
        cpw2 = pltpu.async_copy(w_hbm.at[j_v.at[pl.ds(off, C)]], wj_v, sem_w)
        uvec = u_v[pl.ds(off, L)]
        us = [uvec[r] for r in range(C)]
        rms = [lax.rem(u, 8) for u in us]
        row_cps = []
        for r in range(C):
            g8 = pl.multiple_of(us[r] - rms[r], 8)
            row_cps.append(pltpu.async_copy(
                a_hbm.at[pl.ds(g8, 8), pl.ds(0, DM)], a_v.at[r], sem_a))
            row_cps.append(pltpu.async_copy(
                a_hbm.at[pl.ds(g8, 8), pl.ds(DM, D - DM)], at_v.at[r], sem_a))
        for cp in row_cps:
            cp.wait()
        cpw1.wait()
        cpw2.wait()

        res_i, res_j = chunk_res
        for r in range(C):
            rm = rms[r]

            def k_fn(s, acc, r=r, rm=rm):
                ai, aj = acc
                for t in range(8):
                    av = a_v[r, rm, pl.ds(s * 128 + t * L, L)]
                    ai = ai + av * wi_v[r, s, pl.ds(t * L, L)]
                    aj = aj + av * wj_v[r, s, pl.ds(t * L, L)]
                return ai, aj

            ai, aj = lax.fori_loop(0, 7, k_fn, (zero, zero))
            for t in range(6):
                av = at_v[r, rm, pl.ds(t * L, L)]
                ai = ai + av * wi_v[r, 7, pl.ds(t * L, L)]
                aj = aj + av * wj_v[r, 7, pl.ds(t * L, L)]
            av = at_v[r, rm, pl.ds(88, L)]
            ai = ai + jnp.where(tail_mask, av * wi_v[r, 7, pl.ds(88, L)], 0.0)
            aj = aj + jnp.where(tail_mask, av * wj_v[r, 7, pl.ds(88, L)], 0.0)
            for s in (8, 4, 2, 1):
                perm = lane ^ s
                ai = ai + _permute(ai, perm)
                aj = aj + _permute(aj, perm)
            sel = lane == (r + parity * C)
            res_i = jnp.where(sel, ai, res_i)
            res_j = jnp.where(sel, aj, res_j)

        @pl.when(parity == 1)
        def _store():
            st = (c - 1) * C
            oi_v[pl.ds(st, 2 * C)] = res_i
            oj_v[pl.ds(st, 2 * C)] = res_j

        keep = parity == 0
        return (jnp.where(keep, res_i, zero), jnp.where(keep, res_j, zero))

    lax.fori_loop(0, NCHUNK, chunk_fn, (zero, zero))
    pltpu.sync_copy(oi_v, oi_hbm.at[pl.ds(base, BPW)])
    pltpu.sync_copy(oj_v, oj_hbm.at[pl.ds(base, BPW)])


def kernel(A, W, user, item_i, item_j):
    user = user.astype(jnp.int32)
    item_i = item_i.astype(jnp.int32)
    item_j = item_j.astype(jnp.int32)
    w_pad = _make_w3(W)
    mesh = plsc.VectorSubcoreMesh(core_axis_name="c", subcore_axis_name="s")
    f32 = jnp.float32
    run = pl.kernel(
        _sc_body,
        out_type=(jax.ShapeDtypeStruct((BATCH,), f32),
                  jax.ShapeDtypeStruct((BATCH,), f32)),
        mesh=mesh,
        scratch_types=[
            pltpu.VMEM((BPW + L - C,), jnp.int32),
            pltpu.VMEM((BPW,), jnp.int32),
            pltpu.VMEM((BPW,), jnp.int32),
            pltpu.VMEM((C, 8, DM), f32),
            pltpu.VMEM((C, 8, D - DM), f32),
            pltpu.VMEM((C, 8, 128), f32),
            pltpu.VMEM((C, 8, 128), f32),
            pltpu.VMEM((BPW,), f32),
            pltpu.VMEM((BPW,), f32),
            pltpu.SemaphoreType.DMA,
            pltpu.SemaphoreType.DMA,
        ],
    )
    return run(A, w_pad, user, item_i, item_j)


# speedup vs baseline: 1.4689x; 1.2323x over previous
import functools

import jax
import jax.numpy as jnp
from jax import lax
from jax.experimental import pallas as pl
from jax.experimental.pallas import tpu as pltpu
from jax.experimental.pallas import tpu_sc as plsc

BATCH = 4096
D = 1000
DP = 1024
L = 16
NC, NS = 2, 16
NW = NC * NS
BPW = BATCH // NW
C = 8
NCHUNK = BPW // C
NV = D // L
TAIL = D - NV * L
DM = 896
NGRP = 100000 // 8

_GATHER_DNUMS = lax.GatherDimensionNumbers(
    offset_dims=(), collapsed_slice_dims=(0,), start_index_map=(0,))

TR_BLK = 1024


def _tr_body(at_ref, o_ref):
    o_ref[...] = at_ref[...].T


def _transpose_a(A):
    at = A.T
    return pl.pallas_call(
        _tr_body,
        grid=((100000 + TR_BLK - 1) // TR_BLK,),
        in_specs=[pl.BlockSpec((D, TR_BLK), lambda i: (0, i))],
        out_specs=pl.BlockSpec((TR_BLK, D), lambda i: (i, 0)),
        out_shape=jax.ShapeDtypeStruct((100000, D), jnp.float32),
    )(at)


def _w3_body(w_ref, o_ref):
    for s in range(7):
        o_ref[:, s, :] = w_ref[:, pl.ds(s * 128, 128)]
    o_ref[:, 7, pl.ds(0, D - DM)] = w_ref[:, pl.ds(DM, D - DM)]


def _make_w3(W):
    return pl.pallas_call(
        _w3_body,
        out_shape=jax.ShapeDtypeStruct((D, 8, 128), jnp.float32),
    )(W)


def _permute(v, idx):
    return lax.gather(v, idx[:, None], _GATHER_DNUMS, (1,),
                      mode=lax.GatherScatterMode.PROMISE_IN_BOUNDS)


def _sc_body(a_hbm, w_hbm, u_hbm, i_hbm, j_hbm, oi_hbm, oj_hbm,
             u_v, i_v, j_v, a_v, at_v, wi_v, wj_v, oi_v, oj_v,
             sem_a, sem_w):
    wid = lax.axis_index("s") * NC + lax.axis_index("c")
    base = wid * BPW
    pltpu.sync_copy(u_hbm.at[pl.ds(base, BPW)], u_v.at[pl.ds(0, BPW)])
    pltpu.sync_copy(i_hbm.at[pl.ds(base, BPW)], i_v)
    pltpu.sync_copy(j_hbm.at[pl.ds(base, BPW)], j_v)

    lane = lax.iota(jnp.int32, L)
    tail_mask = lane >= (L - TAIL)
    zero = jnp.zeros((L,), jnp.float32)

    def chunk_fn(c, chunk_res):
        off = c * C
        parity = lax.rem(c, 2)
        cpw1 = pltpu.async_copy(w_hbm.at[i_v.at[pl.ds(off, C)]], wi_v, sem_w)
        cpw2 = pltpu.async_copy(w_hbm.at[j_v.at[pl.ds(off, C)]], wj_v, sem_w)
        uvec = u_v[pl.ds(off, L)]
        us = [uvec[r] for r in range(C)]
        rms = [lax.rem(u, 8) for u in us]
        row_cps = []
        for r in range(C):
            g8 = pl.multiple_of(us[r] - rms[r], 8)
            row_cps.append(pltpu.async_copy(
                a_hbm.at[pl.ds(g8, 8), pl.ds(0, DM)], a_v.at[r], sem_a))
            row_cps.append(pltpu.async_copy(
                a_hbm.at[pl.ds(g8, 8), pl.ds(DM, D - DM)], at_v.at[r], sem_a))
        for cp in row_cps:
            cp.wait()
        cpw1.wait()
        cpw2.wait()

        res_i, res_j = chunk_res
        for r in range(C):
            rm = rms[r]

            def k_fn(s, acc, r=r, rm=rm):
                ai, aj = acc
                for t in range(8):
                    av = a_v[r, rm, pl.ds(s * 128 + t * L, L)]
                    ai = ai + av * wi_v[r, s, pl.ds(t * L, L)]
                    aj = aj + av * wj_v[r, s, pl.ds(t * L, L)]
                return ai, aj

            ai, aj = lax.fori_loop(0, 7, k_fn, (zero, zero))
            for t in range(6):
                av = at_v[r, rm, pl.ds(t * L, L)]
                ai = ai + av * wi_v[r, 7, pl.ds(t * L, L)]
                aj = aj + av * wj_v[r, 7, pl.ds(t * L, L)]
            av = at_v[r, rm, pl.ds(88, L)]
            ai = ai + jnp.where(tail_mask, av * wi_v[r, 7, pl.ds(88, L)], 0.0)
            aj = aj + jnp.where(tail_mask, av * wj_v[r, 7, pl.ds(88, L)], 0.0)
            for s in (8, 4, 2, 1):
                perm = lane ^ s
                ai = ai + _permute(ai, perm)
                aj = aj + _permute(aj, perm)
            sel = lane == (r + parity * C)
            res_i = jnp.where(sel, ai, res_i)
            res_j = jnp.where(sel, aj, res_j)

        @pl.when(parity == 1)
        def _store():
            st = (c - 1) * C
            oi_v[pl.ds(st, 2 * C)] = res_i
            oj_v[pl.ds(st, 2 * C)] = res_j

        keep = parity == 0
        return (jnp.where(keep, res_i, zero), jnp.where(keep, res_j, zero))

    lax.fori_loop(0, NCHUNK, chunk_fn, (zero, zero))
    pltpu.sync_copy(oi_v, oi_hbm.at[pl.ds(base, BPW)])
    pltpu.sync_copy(oj_v, oj_hbm.at[pl.ds(base, BPW)])


def kernel(A, W, user, item_i, item_j):
    user = user.astype(jnp.int32)
    item_i = item_i.astype(jnp.int32)
    item_j = item_j.astype(jnp.int32)
    w_pad = _make_w3(W)
    A = _transpose_a(A)
    mesh = plsc.VectorSubcoreMesh(core_axis_name="c", subcore_axis_name="s")
    f32 = jnp.float32
    run = pl.kernel(
        _sc_body,
        out_type=(jax.ShapeDtypeStruct((BATCH,), f32),
                  jax.ShapeDtypeStruct((BATCH,), f32)),
        mesh=mesh,
        scratch_types=[
            pltpu.VMEM((BPW + L - C,), jnp.int32),
            pltpu.VMEM((BPW,), jnp.int32),
            pltpu.VMEM((BPW,), jnp.int32),
            pltpu.VMEM((C, 8, DM), f32),
            pltpu.VMEM((C, 8, D - DM), f32),
            pltpu.VMEM((C, 8, 128), f32),
            pltpu.VMEM((C, 8, 128), f32),
            pltpu.VMEM((BPW,), f32),
            pltpu.VMEM((BPW,), f32),
            pltpu.SemaphoreType.DMA,
            pltpu.SemaphoreType.DMA,
        ],
    )
    return run(A, w_pad, user, item_i, item_j)


# speedup vs baseline: 1.4991x; 1.0205x over previous
import functools

import jax
import jax.numpy as jnp
from jax import lax
from jax.experimental import pallas as pl
from jax.experimental.pallas import tpu as pltpu
from jax.experimental.pallas import tpu_sc as plsc

BATCH = 4096
D = 1000
DP = 1024
L = 16
NC, NS = 2, 16
NW = NC * NS
BPW = BATCH // NW
C = 8
NCHUNK = BPW // C
NV = D // L
TAIL = D - NV * L
DM = 896
NGRP = 100000 // 8

_GATHER_DNUMS = lax.GatherDimensionNumbers(
    offset_dims=(), collapsed_slice_dims=(0,), start_index_map=(0,))

TR_BLK = 2048


def _tr_body(at_ref, o_ref):
    o_ref[...] = at_ref[...].T


def _transpose_a(A):
    at = A.T
    return pl.pallas_call(
        _tr_body,
        grid=((100000 + TR_BLK - 1) // TR_BLK,),
        in_specs=[pl.BlockSpec((D, TR_BLK), lambda i: (0, i))],
        out_specs=pl.BlockSpec((TR_BLK, D), lambda i: (i, 0)),
        out_shape=jax.ShapeDtypeStruct((100000, D), jnp.float32),
    )(at)


def _w3_body(w_ref, o_ref):
    for s in range(7):
        o_ref[:, s, :] = w_ref[:, pl.ds(s * 128, 128)]
    o_ref[:, 7, pl.ds(0, D - DM)] = w_ref[:, pl.ds(DM, D - DM)]


def _make_w3(W):
    return pl.pallas_call(
        _w3_body,
        out_shape=jax.ShapeDtypeStruct((D, 8, 128), jnp.float32),
    )(W)


def _permute(v, idx):
    return lax.gather(v, idx[:, None], _GATHER_DNUMS, (1,),
                      mode=lax.GatherScatterMode.PROMISE_IN_BOUNDS)


def _sc_body(a_hbm, w_hbm, u_hbm, i_hbm, j_hbm, oi_hbm, oj_hbm,
             u_v, i_v, j_v, a_v, at_v, wi_v, wj_v, oi_v, oj_v,
             sem_a, sem_w):
    wid = lax.axis_index("s") * NC + lax.axis_index("c")
    base = wid * BPW
    pltpu.sync_copy(u_hbm.at[pl.ds(base, BPW)], u_v.at[pl.ds(0, BPW)])
    pltpu.sync_copy(i_hbm.at[pl.ds(base, BPW)], i_v)
    pltpu.sync_copy(j_hbm.at[pl.ds(base, BPW)], j_v)

    lane = lax.iota(jnp.int32, L)
    tail_mask = lane >= (L - TAIL)
    zero = jnp.zeros((L,), jnp.float32)

    def chunk_fn(c, chunk_res):
        off = c * C
        parity = lax.rem(c, 2)
        cpw1 = pltpu.async_copy(w_hbm.at[i_v.at[pl.ds(off, C)]], wi_v, sem_w)
        cpw2 = pltpu.async_copy(w_hbm.at[j_v.at[pl.ds(off, C)]], wj_v, sem_w)
        uvec = u_v[pl.ds(off, L)]
        us = [uvec[r] for r in range(C)]
        rms = [lax.rem(u, 8) for u in us]
        row_cps = []
        for r in range(C):
            g8 = pl.multiple_of(us[r] - rms[r], 8)
            row_cps.append(pltpu.async_copy(
                a_hbm.at[pl.ds(g8, 8), pl.ds(0, DM)], a_v.at[r], sem_a))
            row_cps.append(pltpu.async_copy(
                a_hbm.at[pl.ds(g8, 8), pl.ds(DM, D - DM)], at_v.at[r], sem_a))
        for cp in row_cps:
            cp.wait()
        cpw1.wait()
        cpw2.wait()

        res_i, res_j = chunk_res
        for r in range(C):
            rm = rms[r]

            def k_fn(s, acc, r=r, rm=rm):
                ai, aj = acc
                for t in range(8):
                    av = a_v[r, rm, pl.ds(s * 128 + t * L, L)]
                    ai = ai + av * wi_v[r, s, pl.ds(t * L, L)]
                    aj = aj + av * wj_v[r, s, pl.ds(t * L, L)]
                return ai, aj

            ai, aj = lax.fori_loop(0, 7, k_fn, (zero, zero))
            for t in range(6):
                av = at_v[r, rm, pl.ds(t * L, L)]
                ai = ai + av * wi_v[r, 7, pl.ds(t * L, L)]
                aj = aj + av * wj_v[r, 7, pl.ds(t * L, L)]
            av = at_v[r, rm, pl.ds(88, L)]
            ai = ai + jnp.where(tail_mask, av * wi_v[r, 7, pl.ds(88, L)], 0.0)
            aj = aj + jnp.where(tail_mask, av * wj_v[r, 7, pl.ds(88, L)], 0.0)
            for s in (8, 4, 2, 1):
                perm = lane ^ s
                ai = ai + _permute(ai, perm)
                aj = aj + _permute(aj, perm)
            sel = lane == (r + parity * C)
            res_i = jnp.where(sel, ai, res_i)
            res_j = jnp.where(sel, aj, res_j)

        @pl.when(parity == 1)
        def _store():
            st = (c - 1) * C
            oi_v[pl.ds(st, 2 * C)] = res_i
            oj_v[pl.ds(st, 2 * C)] = res_j

        keep = parity == 0
        return (jnp.where(keep, res_i, zero), jnp.where(keep, res_j, zero))

    lax.fori_loop(0, NCHUNK, chunk_fn, (zero, zero))
    pltpu.sync_copy(oi_v, oi_hbm.at[pl.ds(base, BPW)])
    pltpu.sync_copy(oj_v, oj_hbm.at[pl.ds(base, BPW)])


def kernel(A, W, user, item_i, item_j):
    user = user.astype(jnp.int32)
    item_i = item_i.astype(jnp.int32)
    item_j = item_j.astype(jnp.int32)
    w_pad = _make_w3(W)
    A = _transpose_a(A)
    mesh = plsc.VectorSubcoreMesh(core_axis_name="c", subcore_axis_name="s")
    f32 = jnp.float32
    run = pl.kernel(
        _sc_body,
        out_type=(jax.ShapeDtypeStruct((BATCH,), f32),
                  jax.ShapeDtypeStruct((BATCH,), f32)),
        mesh=mesh,
        scratch_types=[
            pltpu.VMEM((BPW + L - C,), jnp.int32),
            pltpu.VMEM((BPW,), jnp.int32),
            pltpu.VMEM((BPW,), jnp.int32),
            pltpu.VMEM((C, 8, DM), f32),
            pltpu.VMEM((C, 8, D - DM), f32),
            pltpu.VMEM((C, 8, 128), f32),
            pltpu.VMEM((C, 8, 128), f32),
            pltpu.VMEM((BPW,), f32),
            pltpu.VMEM((BPW,), f32),
            pltpu.SemaphoreType.DMA,
            pltpu.SemaphoreType.DMA,
        ],
    )
    return run(A, w_pad, user, item_i, item_j)


# speedup vs baseline: 1.5615x; 1.0416x over previous
import functools

import jax
import jax.numpy as jnp
from jax import lax
from jax.experimental import pallas as pl
from jax.experimental.pallas import tpu as pltpu
from jax.experimental.pallas import tpu_sc as plsc

BATCH = 4096
D = 1000
DP = 1024
L = 16
NC, NS = 2, 16
NW = NC * NS
BPW = BATCH // NW
C = 8
HC = 4
NCHUNK = BPW // C
NV = D // L
TAIL = D - NV * L
DM = 896
NGRP = 100000 // 8

_GATHER_DNUMS = lax.GatherDimensionNumbers(
    offset_dims=(), collapsed_slice_dims=(0,), start_index_map=(0,))

TR_BLK = 2048


def _tr_body(at_ref, o_ref):
    o_ref[...] = at_ref[...].T


def _transpose_a(A):
    at = A.T
    return pl.pallas_call(
        _tr_body,
        grid=((100000 + TR_BLK - 1) // TR_BLK,),
        in_specs=[pl.BlockSpec((D, TR_BLK), lambda i: (0, i))],
        out_specs=pl.BlockSpec((TR_BLK, D), lambda i: (i, 0)),
        out_shape=jax.ShapeDtypeStruct((100000, D), jnp.float32),
    )(at)


def _w3_body(w_ref, o_ref):
    for s in range(7):
        o_ref[:, s, :] = w_ref[:, pl.ds(s * 128, 128)]
    o_ref[:, 7, pl.ds(0, D - DM)] = w_ref[:, pl.ds(DM, D - DM)]


def _make_w3(W):
    return pl.pallas_call(
        _w3_body,
        out_shape=jax.ShapeDtypeStruct((D, 8, 128), jnp.float32),
    )(W)


def _permute(v, idx):
    return lax.gather(v, idx[:, None], _GATHER_DNUMS, (1,),
                      mode=lax.GatherScatterMode.PROMISE_IN_BOUNDS)


def _sc_body(a_hbm, w_hbm, u_hbm, i_hbm, j_hbm, oi_hbm, oj_hbm,
             u_v, i_v, j_v, a_v, at_v, wi_v, wj_v, oi_v, oj_v,
             sem_a0, sem_a1, sem_w0, sem_w1):
    wid = lax.axis_index("s") * NC + lax.axis_index("c")
    base = wid * BPW
    pltpu.sync_copy(u_hbm.at[pl.ds(base, BPW)], u_v.at[pl.ds(0, BPW)])
    pltpu.sync_copy(i_hbm.at[pl.ds(base, BPW)], i_v)
    pltpu.sync_copy(j_hbm.at[pl.ds(base, BPW)], j_v)

    lane = lax.iota(jnp.int32, L)
    tail_mask = lane >= (L - TAIL)
    zero = jnp.zeros((L,), jnp.float32)
    sems_w = (sem_w0, sem_w1)
    sems_a = (sem_a0, sem_a1)

    def chunk_scalars(cc):
        uvec = u_v[pl.ds(cc * C, L)]
        us = [uvec[r] for r in range(C)]
        rms = [lax.rem(u, 8) for u in us]
        g8s = [pl.multiple_of(us[r] - rms[r], 8) for r in range(C)]
        return g8s, rms

    def issue_w(cc, ws):
        off = cc * C
        pltpu.async_copy(w_hbm.at[i_v.at[pl.ds(off, C)]], wi_v.at[ws],
                         sems_w[ws])
        pltpu.async_copy(w_hbm.at[j_v.at[pl.ds(off, C)]], wj_v.at[ws],
                         sems_w[ws])
        g8s, _ = chunk_scalars(cc)
        for r in range(C):
            pltpu.async_copy(a_hbm.at[pl.ds(g8s[r], 8), pl.ds(DM, D - DM)],
                             at_v.at[ws, r], sems_w[ws])

    def issue_a(cc, half):
        g8s, _ = chunk_scalars(cc)
        for r in range(half * HC, half * HC + HC):
            pltpu.async_copy(a_hbm.at[pl.ds(g8s[r], 8), pl.ds(0, DM)],
                             a_v.at[r], sems_a[half])

    def wait_w(ws):
        pltpu.make_async_copy(w_hbm.at[pl.ds(0, C)], wi_v.at[ws],
                              sems_w[ws]).wait()
        pltpu.make_async_copy(w_hbm.at[pl.ds(0, C)], wj_v.at[ws],
                              sems_w[ws]).wait()
        for r in range(C):
            pltpu.make_async_copy(
                a_hbm.at[pl.ds(0, 8), pl.ds(DM, D - DM)], at_v.at[ws, r],
                sems_w[ws]).wait()

    def wait_a(half):
        for r in range(half * HC, half * HC + HC):
            pltpu.make_async_copy(
                a_hbm.at[pl.ds(0, 8), pl.ds(0, DM)], a_v.at[r],
                sems_a[half]).wait()

    def compute_rows(cc, half, ws, parity, res):
        res_i, res_j = res
        _, rms = chunk_scalars(cc)
        for r in range(half * HC, half * HC + HC):
            rm = rms[r]

            def k_fn(t7, acc, r=r, rm=rm):
                ai, aj = acc
                for t in range(8):
                    av = a_v[r, rm, pl.ds(t7 * 128 + t * L, L)]
                    ai = ai + av * wi_v[ws, r, t7, pl.ds(t * L, L)]
                    aj = aj + av * wj_v[ws, r, t7, pl.ds(t * L, L)]
                return ai, aj

            ai, aj = lax.fori_loop(0, 7, k_fn, (zero, zero))
            for t in range(6):
                av = at_v[ws, r, rm, pl.ds(t * L, L)]
                ai = ai + av * wi_v[ws, r, 7, pl.ds(t * L, L)]
                aj = aj + av * wj_v[ws, r, 7, pl.ds(t * L, L)]
            av = at_v[ws, r, rm, pl.ds(88, L)]
            ai = ai + jnp.where(tail_mask,
                                av * wi_v[ws, r, 7, pl.ds(88, L)], 0.0)
            aj = aj + jnp.where(tail_mask,
                                av * wj_v[ws, r, 7, pl.ds(88, L)], 0.0)
            for sft in (8, 4, 2, 1):
                perm = lane ^ sft
                ai = ai + _permute(ai, perm)
                aj = aj + _permute(aj, perm)
            sel = lane == (r + parity * C)
            res_i = jnp.where(sel, ai, res_i)
            res_j = jnp.where(sel, aj, res_j)
        return res_i, res_j

    def half_chunk(cc, ws, parity, res):
        wait_w(ws)
        wait_a(0)
        res = compute_rows(cc, 0, ws, parity, res)

        @pl.when(cc + 1 < NCHUNK)
        def _():
            issue_a(cc + 1, 0)

        wait_a(1)
        res = compute_rows(cc, 1, ws, parity, res)

        @pl.when(cc + 1 < NCHUNK)
        def _():
            issue_a(cc + 1, 1)

        @pl.when(cc + 2 < NCHUNK)
        def _():
            issue_w(cc + 2, ws)

        return res

    issue_w(0, 0)
    issue_w(1, 1)
    issue_a(0, 0)
    issue_a(0, 1)

    def pair_fn(c2, _):
        ca = c2 * 2
        res = half_chunk(ca, 0, 0, (zero, zero))
        res = half_chunk(ca + 1, 1, 1, res)
        oi_v[pl.ds(ca * C, 2 * C)] = res[0]
        oj_v[pl.ds(ca * C, 2 * C)] = res[1]
        return 0

    lax.fori_loop(0, NCHUNK // 2, pair_fn, 0)
    pltpu.sync_copy(oi_v, oi_hbm.at[pl.ds(base, BPW)])
    pltpu.sync_copy(oj_v, oj_hbm.at[pl.ds(base, BPW)])


def kernel(A, W, user, item_i, item_j):
    user = user.astype(jnp.int32)
    item_i = item_i.astype(jnp.int32)
    item_j = item_j.astype(jnp.int32)
    w_pad = _make_w3(W)
    A = _transpose_a(A)
    mesh = plsc.VectorSubcoreMesh(core_axis_name="c", subcore_axis_name="s")
    f32 = jnp.float32
    run = pl.kernel(
        _sc_body,
        out_type=(jax.ShapeDtypeStruct((BATCH,), f32),
                  jax.ShapeDtypeStruct((BATCH,), f32)),
        mesh=mesh,
        scratch_types=[
            pltpu.VMEM((BPW + L - C,), jnp.int32),
            pltpu.VMEM((BPW,), jnp.int32),
            pltpu.VMEM((BPW,), jnp.int32),
            pltpu.VMEM((C, 8, DM), f32),
            pltpu.VMEM((2, C, 8, D - DM), f32),
            pltpu.VMEM((2, C, 8, 128), f32),
            pltpu.VMEM((2, C, 8, 128), f32),
            pltpu.VMEM((BPW,), f32),
            pltpu.VMEM((BPW,), f32),
            pltpu.SemaphoreType.DMA,
            pltpu.SemaphoreType.DMA,
            pltpu.SemaphoreType.DMA,
            pltpu.SemaphoreType.DMA,
        ],
    )
    return run(A, w_pad, user, item_i, item_j)
